# baseline (speedup 1.0000x reference)
import jax
import jax.numpy as jnp
from jax.experimental import pallas as pl
from jax.experimental.pallas import tpu as pltpu

_GB = 32


def _gconv_kernel(x_ref, w_ref, b_ref, o_ref):
    for g in range(_GB):
        xg = x_ref[g]
        wg = w_ref[g]
        og = jax.lax.dot_general(
            xg, wg,
            dimension_numbers=(((1,), (1,)), ((), ())),
            preferred_element_type=jnp.float32,
        )
        o_ref[g] = og + b_ref[g : g + 1, :]


def kernel(x, W, b):
    B, Cin, H, Wsp = x.shape
    G = H * Wsp
    Cout = W.shape[1]
    xg = jnp.transpose(x, (2, 3, 0, 1)).reshape(G, B, Cin)
    out = pl.pallas_call(
        _gconv_kernel,
        grid=(G // _GB,),
        in_specs=[
            pl.BlockSpec((_GB, B, Cin), lambda j: (j, 0, 0)),
            pl.BlockSpec((_GB, Cout, Cin), lambda j: (j, 0, 0)),
            pl.BlockSpec((_GB, Cout), lambda j: (j, 0)),
        ],
        out_specs=pl.BlockSpec((_GB, B, Cout), lambda j: (j, 0, 0)),
        out_shape=jax.ShapeDtypeStruct((G, B, Cout), jnp.float32),
        compiler_params=pltpu.CompilerParams(
            dimension_semantics=("parallel",),
        ),
        name="regroup_conv_v1",
    )(xg, W, b)
    return jnp.transpose(out, (1, 2, 0)).reshape(B, Cout, H, Wsp)
